# baseline (speedup 1.0000x reference)
import jax
import jax.numpy as jnp
from jax.experimental import pallas as pl
from jax.experimental.pallas import tpu as pltpu

T = 32768
V = 1024
BLANK = V - 1
BT = 512
NBLK = T // BT


def _body(x_ref, idx_ref, keep_ref, score_ref, prev_ref):
    i = pl.program_id(0)

    @pl.when(i == 0)
    def _():
        prev_ref[0] = -1

    x = x_ref[...]
    m = jnp.max(x, axis=-1, keepdims=True)
    lane = jax.lax.broadcasted_iota(jnp.int32, x.shape, 1)
    cand = jnp.where(x == m, lane, V)
    idx = jnp.min(cand, axis=-1)
    idx2 = idx.reshape(1, BT)
    prev_first = jnp.full((1, 1), prev_ref[0], dtype=jnp.int32)
    prev = jnp.concatenate([prev_first, idx2[:, : BT - 1]], axis=1)
    keep = (idx2 != BLANK) & (idx2 != prev)
    idx_ref[0, :, :] = idx2
    keep_ref[0, :, :] = keep.astype(jnp.int32)
    score_ref[0, :, :] = m.reshape(1, BT)
    prev_ref[0] = idx2[0, BT - 1]


def kernel(emission):
    idx3, keep3, score3 = pl.pallas_call(
        _body,
        grid=(NBLK,),
        in_specs=[pl.BlockSpec((BT, V), lambda i: (i, 0))],
        out_specs=[
            pl.BlockSpec((1, 1, BT), lambda i: (i, 0, 0)),
            pl.BlockSpec((1, 1, BT), lambda i: (i, 0, 0)),
            pl.BlockSpec((1, 1, BT), lambda i: (i, 0, 0)),
        ],
        out_shape=[
            jax.ShapeDtypeStruct((NBLK, 1, BT), jnp.int32),
            jax.ShapeDtypeStruct((NBLK, 1, BT), jnp.int32),
            jax.ShapeDtypeStruct((NBLK, 1, BT), jnp.float32),
        ],
        scratch_shapes=[pltpu.SMEM((1,), jnp.int32)],
    )(emission)
    idx = idx3.reshape(T)
    keep = keep3.reshape(T).astype(bool)
    scores = score3.reshape(T)
    return idx, keep, scores


# speedup vs baseline: 1.2617x; 1.2617x over previous
import jax
import jax.numpy as jnp
from jax.experimental import pallas as pl
from jax.experimental.pallas import tpu as pltpu

T = 32768
V = 1024
BLANK = V - 1
BT = 512
NBLK = T // BT


def _body(x_ref, idx_ref, keep_ref, score_ref, prev_ref):
    i = pl.program_id(0)

    @pl.when(i == 0)
    def _():
        prev_ref[0] = -1

    v = x_ref[:, 0:128]
    bestc = jnp.zeros((BT, 128), jnp.int32)
    for c in range(1, 8):
        u = x_ref[:, c * 128 : (c + 1) * 128]
        gt = u > v
        v = jnp.where(gt, u, v)
        bestc = jnp.where(gt, c, bestc)
    vT = v.T
    cT = bestc.T
    m = jnp.max(vT, axis=0)
    lane0 = jax.lax.broadcasted_iota(jnp.int32, (128, BT), 0)
    posT = cT * 128 + lane0
    cand = jnp.where(vT == m[None, :], posT, V)
    idx = jnp.min(cand, axis=0)
    idx2 = idx.reshape(1, BT)
    prev_first = jnp.full((1, 1), prev_ref[0], dtype=jnp.int32)
    prev = jnp.concatenate([prev_first, idx2[:, : BT - 1]], axis=1)
    keep = (idx2 != BLANK) & (idx2 != prev)
    idx_ref[0, :, :] = idx2
    keep_ref[0, :, :] = keep.astype(jnp.int32)
    score_ref[0, :, :] = m.reshape(1, BT)
    prev_ref[0] = idx2[0, BT - 1]


def kernel(emission):
    idx3, keep3, score3 = pl.pallas_call(
        _body,
        grid=(NBLK,),
        in_specs=[pl.BlockSpec((BT, V), lambda i: (i, 0))],
        out_specs=[
            pl.BlockSpec((1, 1, BT), lambda i: (i, 0, 0)),
            pl.BlockSpec((1, 1, BT), lambda i: (i, 0, 0)),
            pl.BlockSpec((1, 1, BT), lambda i: (i, 0, 0)),
        ],
        out_shape=[
            jax.ShapeDtypeStruct((NBLK, 1, BT), jnp.int32),
            jax.ShapeDtypeStruct((NBLK, 1, BT), jnp.int32),
            jax.ShapeDtypeStruct((NBLK, 1, BT), jnp.float32),
        ],
        scratch_shapes=[pltpu.SMEM((1,), jnp.int32)],
    )(emission)
    idx = idx3.reshape(T)
    keep = keep3.reshape(T).astype(bool)
    scores = score3.reshape(T)
    return idx, keep, scores
